# baseline (speedup 1.0000x reference)
import functools

import jax
import jax.numpy as jnp
from jax import lax
from jax.experimental import pallas as pl
from jax.experimental.pallas import tpu as pltpu
from jax.experimental.pallas import tpu_sc as plsc

D_MODEL = 128
SCALE = float(D_MODEL) ** 0.5
LANES = 16
NUM_CORES = 2
NUM_SUBCORES = 16
NUM_WORKERS = NUM_CORES * NUM_SUBCORES
CHUNK = 128


def _make_lookup(batch: int):
    assert batch % (NUM_WORKERS * CHUNK) == 0
    per_worker = batch // NUM_WORKERS
    n_chunks = per_worker // CHUNK

    mesh = plsc.VectorSubcoreMesh(core_axis_name="c", subcore_axis_name="s")

    @functools.partial(
        pl.kernel,
        mesh=mesh,
        out_type=jax.ShapeDtypeStruct((batch, D_MODEL), jnp.float32),
        scratch_types=[
            pltpu.VMEM((n_chunks, CHUNK), jnp.int32),
            pltpu.VMEM((2, CHUNK, D_MODEL), jnp.float32),
            pltpu.SemaphoreType.DMA,
            pltpu.SemaphoreType.DMA,
            pltpu.SemaphoreType.DMA,
            pltpu.SemaphoreType.DMA,
        ],
    )
    def lookup(ids_hbm, table_hbm, out_hbm, idx_v, rows_v, g0, g1, s0, s1):
        gsem = (g0, g1)
        ssem = (s0, s1)
        wid = lax.axis_index("s") * NUM_CORES + lax.axis_index("c")
        base = wid * per_worker
        pltpu.sync_copy(ids_hbm.at[pl.ds(wid * n_chunks, n_chunks)], idx_v)

        def start_gather(c, b):
            return pltpu.async_copy(
                table_hbm.at[idx_v.at[c]], rows_v.at[b], gsem[b]
            )

        gathers = [None] * n_chunks
        scatters = [None] * n_chunks
        gathers[0] = start_gather(0, 0)
        for c in range(n_chunks):
            b = c & 1
            if c >= 1:
                scatters[c - 1].wait()
            if c + 1 < n_chunks:
                gathers[c + 1] = start_gather(c + 1, 1 - b)
            gathers[c].wait()

            @plsc.parallel_loop(0, CHUNK, step=1, unroll=4)
            def scale_row(r):
                for j in range(D_MODEL // LANES):
                    sl = pl.ds(j * LANES, LANES)
                    rows_v[b, r, sl] = rows_v[b, r, sl] * SCALE
            scatters[c] = pltpu.async_copy(
                rows_v.at[b], out_hbm.at[pl.ds(base + c * CHUNK, CHUNK)], ssem[b]
            )
        scatters[n_chunks - 1].wait()

    return lookup


def kernel(input_ids, table):
    b0, b1 = input_ids.shape
    batch = b0 * b1
    ids2d = input_ids.reshape(batch // CHUNK, CHUNK).astype(jnp.int32)
    out = _make_lookup(batch)(ids2d, table)
    return out.reshape(b0, b1, D_MODEL)


# speedup vs baseline: 1.0707x; 1.0707x over previous
import functools

import jax
import jax.numpy as jnp
from jax import lax
from jax.experimental import pallas as pl
from jax.experimental.pallas import tpu as pltpu
from jax.experimental.pallas import tpu_sc as plsc

D_MODEL = 128
SCALE = float(D_MODEL) ** 0.5
LANES = 16
NUM_CORES = 2
NUM_SUBCORES = 16
NUM_WORKERS = NUM_CORES * NUM_SUBCORES
CHUNK = 128
NBUF = 4


def _make_lookup(batch: int):
    assert batch % (NUM_WORKERS * CHUNK) == 0
    per_worker = batch // NUM_WORKERS
    n_chunks = per_worker // CHUNK

    mesh = plsc.VectorSubcoreMesh(core_axis_name="c", subcore_axis_name="s")

    @functools.partial(
        pl.kernel,
        mesh=mesh,
        out_type=jax.ShapeDtypeStruct((batch, D_MODEL), jnp.float32),
        scratch_types=[
            pltpu.VMEM((n_chunks, CHUNK), jnp.int32),
            pltpu.VMEM((NBUF, CHUNK, D_MODEL), jnp.float32),
        ]
        + [pltpu.SemaphoreType.DMA] * (2 * NBUF),
    )
    def lookup(ids_hbm, table_hbm, out_hbm, idx_v, rows_v, *sems):
        gsem = sems[:NBUF]
        ssem = sems[NBUF:]
        wid = lax.axis_index("s") * NUM_CORES + lax.axis_index("c")
        base = wid * per_worker
        pltpu.sync_copy(ids_hbm.at[pl.ds(wid * n_chunks, n_chunks)], idx_v)

        def start_gather(c):
            b = c % NBUF
            return pltpu.async_copy(
                table_hbm.at[idx_v.at[c]], rows_v.at[b], gsem[b]
            )

        gathers = [None] * n_chunks
        scatters = [None] * n_chunks
        for c in range(min(NBUF - 1, n_chunks)):
            gathers[c] = start_gather(c)
        for c in range(n_chunks):
            b = c % NBUF
            if c + NBUF - 1 < n_chunks:
                if c >= 1:
                    scatters[c - 1].wait()
                gathers[c + NBUF - 1] = start_gather(c + NBUF - 1)
            gathers[c].wait()

            def scale_row(r, _):
                for j in range(D_MODEL // LANES):
                    sl = pl.ds(j * LANES, LANES)
                    rows_v[b, r, sl] = rows_v[b, r, sl] * SCALE
                return 0

            lax.fori_loop(0, CHUNK, scale_row, 0)
            scatters[c] = pltpu.async_copy(
                rows_v.at[b], out_hbm.at[pl.ds(base + c * CHUNK, CHUNK)], ssem[b]
            )
        for c in range(max(0, n_chunks - NBUF), n_chunks):
            scatters[c].wait()

    return lookup


def kernel(input_ids, table):
    b0, b1 = input_ids.shape
    batch = b0 * b1
    ids2d = input_ids.reshape(batch // CHUNK, CHUNK).astype(jnp.int32)
    out = _make_lookup(batch)(ids2d, table)
    return out.reshape(b0, b1, D_MODEL)


# speedup vs baseline: 1.0755x; 1.0045x over previous
import functools

import jax
import jax.numpy as jnp
from jax import lax
from jax.experimental import pallas as pl
from jax.experimental.pallas import tpu as pltpu
from jax.experimental.pallas import tpu_sc as plsc

D_MODEL = 128
SCALE = float(D_MODEL) ** 0.5
LANES = 16
NUM_CORES = 2
NUM_SUBCORES = 16
NUM_WORKERS = NUM_CORES * NUM_SUBCORES
CHUNK = 64
NBUF = 6


def _make_lookup(batch: int):
    assert batch % (NUM_WORKERS * CHUNK) == 0
    per_worker = batch // NUM_WORKERS
    n_chunks = per_worker // CHUNK

    mesh = plsc.VectorSubcoreMesh(core_axis_name="c", subcore_axis_name="s")

    @functools.partial(
        pl.kernel,
        mesh=mesh,
        out_type=jax.ShapeDtypeStruct((batch, D_MODEL), jnp.float32),
        scratch_types=[
            pltpu.VMEM((n_chunks, CHUNK), jnp.int32),
            pltpu.VMEM((NBUF, CHUNK, D_MODEL), jnp.float32),
        ]
        + [pltpu.SemaphoreType.DMA] * (2 * NBUF),
    )
    def lookup(ids_hbm, table_hbm, out_hbm, idx_v, rows_v, *sems):
        gsem = sems[:NBUF]
        ssem = sems[NBUF:]
        wid = lax.axis_index("s") * NUM_CORES + lax.axis_index("c")
        base = wid * per_worker
        pltpu.sync_copy(ids_hbm.at[pl.ds(wid * n_chunks, n_chunks)], idx_v)

        def start_gather(c):
            b = c % NBUF
            return pltpu.async_copy(
                table_hbm.at[idx_v.at[c]], rows_v.at[b], gsem[b]
            )

        gathers = [None] * n_chunks
        scatters = [None] * n_chunks
        for c in range(min(NBUF - 1, n_chunks)):
            gathers[c] = start_gather(c)
        for c in range(n_chunks):
            b = c % NBUF
            if c + NBUF - 1 < n_chunks:
                if c >= 1:
                    scatters[c - 1].wait()
                gathers[c + NBUF - 1] = start_gather(c + NBUF - 1)
            gathers[c].wait()

            def scale_row(r, _):
                for j in range(D_MODEL // LANES):
                    sl = pl.ds(j * LANES, LANES)
                    rows_v[b, r, sl] = rows_v[b, r, sl] * SCALE
                return 0

            lax.fori_loop(0, CHUNK, scale_row, 0)
            scatters[c] = pltpu.async_copy(
                rows_v.at[b], out_hbm.at[pl.ds(base + c * CHUNK, CHUNK)], ssem[b]
            )
        for c in range(max(0, n_chunks - NBUF), n_chunks):
            scatters[c].wait()

    return lookup


def kernel(input_ids, table):
    b0, b1 = input_ids.shape
    batch = b0 * b1
    ids2d = input_ids.reshape(batch // CHUNK, CHUNK).astype(jnp.int32)
    out = _make_lookup(batch)(ids2d, table)
    return out.reshape(b0, b1, D_MODEL)
